# Initial kernel scaffold; baseline (speedup 1.0000x reference)
import jax
import jax.numpy as jnp
from jax.experimental import pallas as pl


def kernel(x, Wd1, bd1, Wd2, bd2, Wd3, bd3, Pin, Pin_b, codebooks, Pout, Pout_b, Wu1, bu1, Wu2, bu2, Wu3, bu3):
    raise NotImplementedError("write your pallas kernel here")


# speedup vs baseline: 1.4284x; 1.4284x over previous
import functools

import jax
import jax.numpy as jnp
from jax.experimental import pallas as pl
from jax.experimental.pallas import tpu as pltpu


def _body(x_ref, wd1, bd1, wd2, bd2, wd3, bd3, pin, pinb, cb, pout, poutb,
          wu1, bu1, wu2, bu2, wu3, bu3, u_ref, idx_ref):
    f32 = jnp.float32
    xb = x_ref[...]
    h = jnp.maximum(jnp.dot(xb, wd1[...], preferred_element_type=f32) + bd1[...], 0.0)
    h = jnp.maximum(jnp.dot(h, wd2[...], preferred_element_type=f32) + bd2[...], 0.0)
    h = jnp.dot(h, wd3[...], preferred_element_type=f32) + bd3[...]

    E, K, CD = cb.shape
    bB = h.shape[0]
    acc = None
    idx_cols = []
    for i in range(E):
        z = jnp.dot(h, pin[i], preferred_element_type=f32) + pinb[i]
        cbi = cb[i]
        cross = jax.lax.dot_general(z, cbi, (((1,), (1,)), ((), ())),
                                    preferred_element_type=f32)
        d = (jnp.sum(z * z, axis=1, keepdims=True) - 2.0 * cross
             + jnp.sum(cbi * cbi, axis=1)[None, :])
        dmin = jnp.min(d, axis=1, keepdims=True)
        iota = jax.lax.broadcasted_iota(jnp.int32, (bB, K), 1)
        idx = jnp.min(jnp.where(d == dmin, iota, K), axis=1)
        onehot = (iota == idx[:, None]).astype(f32)
        q = jnp.dot(onehot, cbi, preferred_element_type=f32)
        out_i = jnp.dot(q, pout[i], preferred_element_type=f32) + poutb[i]
        acc = out_i if acc is None else acc + out_i
        idx_cols.append(idx)

    hq = acc * (1.0 / E)
    u = jnp.maximum(jnp.dot(hq, wu1[...], preferred_element_type=f32) + bu1[...], 0.0)
    u = jnp.maximum(jnp.dot(u, wu2[...], preferred_element_type=f32) + bu2[...], 0.0)
    u = jnp.dot(u, wu3[...], preferred_element_type=f32) + bu3[...]
    u_ref[...] = jnp.clip(u, -1.0, 1.0)

    idx3 = jnp.stack(idx_cols, axis=1)
    pad = idx_ref.shape[1] - E
    idx_ref[...] = jnp.concatenate(
        [idx3, jnp.zeros((bB, pad), jnp.int32)], axis=1)


def kernel(x, Wd1, bd1, Wd2, bd2, Wd3, bd3, Pin, Pin_b, codebooks, Pout,
           Pout_b, Wu1, bu1, Wu2, bu2, Wu3, bu3):
    B, D = x.shape
    H = Wd3.shape[1]
    E, K, CD = codebooks.shape
    bB = 256
    grid = (B // bB,)
    IDXP = 8

    def full(a):
        return pl.BlockSpec(a.shape, lambda i: (0,) * a.ndim)

    weights = (Wd1, bd1, Wd2, bd2, Wd3, bd3, Pin, Pin_b, codebooks, Pout,
               Pout_b, Wu1, bu1, Wu2, bu2, Wu3, bu3)
    u, idx = pl.pallas_call(
        _body,
        grid=grid,
        in_specs=[pl.BlockSpec((bB, D), lambda i: (i, 0))] +
                 [full(w) for w in weights],
        out_specs=[pl.BlockSpec((bB, D), lambda i: (i, 0)),
                   pl.BlockSpec((bB, IDXP), lambda i: (i, 0))],
        out_shape=[jax.ShapeDtypeStruct((B, D), jnp.float32),
                   jax.ShapeDtypeStruct((B, IDXP), jnp.int32)],
        compiler_params=pltpu.CompilerParams(
            dimension_semantics=("arbitrary",)),
    )(x, *weights)
    return u, idx[:, :E], jnp.zeros((), jnp.float32)
